# baseline (speedup 1.0000x reference)
import functools

import jax
import jax.numpy as jnp
from jax import lax
from jax.experimental import pallas as pl
from jax.experimental.pallas import tpu as pltpu
from jax.experimental.pallas import tpu_sc as plsc

D = 128
N = 1_000_000
B = 16384
NUM_CORES = 2
NUM_SUBCORES = 16
NW = NUM_CORES * NUM_SUBCORES
ROWS_PER_TILE = D // NW
CHUNK = 16384
NCHUNKS = B // CHUNK
LANES = 16


def _sc_gather(mem_flat, idx):
    mesh = plsc.VectorSubcoreMesh(core_axis_name="c", subcore_axis_name="s")

    @functools.partial(
        pl.kernel,
        mesh=mesh,
        out_type=jax.ShapeDtypeStruct((D, B), jnp.float32),
        scratch_types=[
            pltpu.VMEM((B,), jnp.int32),
            pltpu.VMEM((B,), jnp.int32),
            pltpu.VMEM((B,), jnp.float32),
            pltpu.SemaphoreType.DMA,
        ],
    )
    def k(mem_hbm, idx_hbm, out_hbm, idx_v, fidx_v, row_v, sem):
        wid = lax.axis_index("s") * NUM_CORES + lax.axis_index("c")
        pltpu.sync_copy(idx_hbm, idx_v)
        for r in range(ROWS_PER_TILE):
            d = wid * ROWS_PER_TILE + r
            base = d * N

            def add_body(i, carry):
                fidx_v[pl.ds(i * LANES, LANES)] = (
                    idx_v[pl.ds(i * LANES, LANES)] + base
                )
                return carry

            lax.fori_loop(0, B // LANES, add_body, 0)

            def fire(c, carry):
                pltpu.async_copy(
                    mem_hbm.at[fidx_v.at[pl.ds(c * CHUNK, CHUNK)]],
                    row_v.at[pl.ds(c * CHUNK, CHUNK)],
                    sem,
                )
                return carry

            lax.fori_loop(0, NCHUNKS, fire, 0)

            pltpu.make_async_copy(mem_hbm.at[pl.ds(0, B)], row_v, sem).wait()
            pltpu.sync_copy(row_v, out_hbm.at[d])

    return k(mem_flat, idx)


def kernel(membank, n_index):
    mem_flat = membank.reshape(-1)
    idx = n_index.astype(jnp.int32)
    return _sc_gather(mem_flat, idx)


# speedup vs baseline: 14.4261x; 14.4261x over previous
import functools

import jax
import jax.numpy as jnp
from jax import lax
from jax.experimental import pallas as pl
from jax.experimental.pallas import tpu as pltpu
from jax.experimental.pallas import tpu_sc as plsc

DIM = 128
N = 1_000_000
B = 16384
NUM_CORES = 2
NUM_SUBCORES = 16
NW = NUM_CORES * NUM_SUBCORES
NBLK = (N + 127) // 128
BPT = (NBLK + NW - 1) // NW
NSB = 16
STEPS = B // 16
CAP = B + 16 * NSB
SENT = 1 << 27
SC_SPILL = B + NUM_SUBCORES * 64
POS_ROWS = B + NW

S_CNT, S_WROW, S_FROW, S_NFIRE, S_NWAIT, S_GBASE, S_XCHG = 0, 1, 2, 3, 4, 5, 6
S_BASE = 8
S_ESTEP = 24
S_OFFS = 40

_MESH = plsc.VectorSubcoreMesh(core_axis_name="c", subcore_axis_name="s")


@functools.partial(
    pl.kernel,
    mesh=_MESH,
    compiler_params=pltpu.CompilerParams(needs_layout_passes=False),
    out_type=[
        jax.ShapeDtypeStruct((NUM_CORES * SC_SPILL, 128), jnp.float32),
        jax.ShapeDtypeStruct((POS_ROWS,), jnp.int32),
    ],
    scratch_types=[
        pltpu.VMEM((CAP,), jnp.int32),
        pltpu.VMEM((CAP,), jnp.int32),
        pltpu.VMEM((CAP,), jnp.int32),
        pltpu.VMEM((CAP,), jnp.int32),
        pltpu.VMEM((256, 128), jnp.float32),
        pltpu.VMEM((128, 128), jnp.float32),
        pltpu.VMEM((2, 64), jnp.int32),
        pltpu.VMEM((2, 64), jnp.int32),
        pltpu.SMEM((64,), jnp.int32),
        pltpu.SemaphoreType.DMA,
        pltpu.SemaphoreType.DMA,
    ],
)
def _scan_spill(mem_hbm, idx_hbm, spill_hbm, pos_hbm, X, An, Ak, Dk, blk,
                stage, kg, posbuf, scal, sem_b, sem_s):
    scid = lax.axis_index("c")
    sid = lax.axis_index("s")
    wid = sid * NUM_CORES + scid
    cb0 = wid * BPT
    nb = jnp.minimum(BPT, NBLK - cb0)
    lo = cb0 * 128
    hi = jnp.minimum((cb0 + nb) * 128, N)
    iota = lax.iota(jnp.int32, 16)
    pad_row = B + wid

    for i in range(7):
        scal[i] = 0
    for slot in range(2):
        for i in range(4):
            plsc.store_scatter(
                kg,
                [jnp.full((16,), slot, jnp.int32), 16 * i + iota],
                jnp.full((16,), pad_row, jnp.int32),
            )

    def fire_blk(cb):
        slot = jnp.bitwise_and(cb, 1)
        col = pl.multiple_of((cb0 + cb) * 128, 128)
        pltpu.async_copy(
            mem_hbm.at[:, pl.ds(col, 128)],
            blk.at[pl.ds(slot * 128, 128), :],
            sem_b,
        )

    def wait_blk(cb):
        pltpu.make_async_copy(
            mem_hbm.at[:, pl.ds(0, 128)],
            blk.at[pl.ds(0, 128), :],
            sem_b,
        ).wait()

    def drain_one_spill():
        pltpu.make_async_copy(
            spill_hbm.at[pl.ds(0, 64), :],
            stage.at[pl.ds(0, 64), :],
            sem_s,
        ).wait()
        pltpu.make_async_copy(
            idx_hbm.at[pl.ds(0, 64)],
            posbuf.at[0],
            sem_s,
        ).wait()
        scal[S_NWAIT] = scal[S_NWAIT] + 1

    def flush():
        frow = scal[S_FROW]
        slot = jnp.bitwise_and(lax.shift_right_logical(frow, 6), 1)
        dst_row = pl.multiple_of(scal[S_GBASE] + frow, 64)
        grow = jnp.full((16,), dst_row, jnp.int32)
        for i in range(4):
            plsc.store_scatter(
                posbuf,
                [jnp.full((16,), slot, jnp.int32), 16 * i + iota],
                grow + 16 * i + iota,
            )
        pltpu.async_copy(
            stage.at[pl.ds(slot * 64, 64), :],
            spill_hbm.at[pl.ds(dst_row, 64), :],
            sem_s,
        )
        pltpu.async_copy(
            posbuf.at[slot],
            pos_hbm.at[kg.at[slot]],
            sem_s,
        )
        scal[S_FROW] = frow + 64
        scal[S_NFIRE] = scal[S_NFIRE] + 1

        @pl.when(scal[S_NFIRE] - scal[S_NWAIT] >= 2)
        def _():
            drain_one_spill()

    pltpu.sync_copy(idx_hbm.at[pl.ds(0, B)], X.at[pl.ds(0, B)])
    fire_blk(0)

    @pl.when(nb > 1)
    def _():
        fire_blk(1)

    def filt(s, carry):
        v = X[pl.ds(16 * s, 16)]
        m = (v >= lo) & (v < hi)
        pc = plsc.cumsum(m.astype(jnp.int32))
        cnt = scal[S_CNT]
        pos = cnt + pc - 1
        plsc.store_scatter(An, [pos], v, mask=m)
        plsc.store_scatter(Ak, [pos], 16 * s + iota, mask=m)
        scal[S_CNT] = cnt + jnp.max(pc)
        return carry

    lax.fori_loop(0, STEPS, filt, 0)
    cnt = scal[S_CNT]
    nsteps = lax.shift_right_logical(cnt + 15, 4)

    aligned_cnt = lax.shift_left(lax.shift_right_logical(cnt + 63, 6), 6)
    plsc.subcore_barrier()
    my_base = plsc.fetch_and_add(scal.at[S_XCHG], aligned_cnt, subcore_id=0)
    scal[S_GBASE] = scid * SC_SPILL + my_base

    for sb in range(NSB):
        scal[S_OFFS + sb] = 0

    def count_step(u, carry):
        v = An[pl.ds(16 * u, 16)]
        valid = (16 * u + iota) < cnt
        sbv = lax.shift_right_logical(v - lo, 11)
        for sb in range(NSB):
            msb = valid & (sbv == sb)
            pc = plsc.all_reduce_population_count(msb)
            scal[S_OFFS + sb] = scal[S_OFFS + sb] + jnp.max(pc)
        return carry

    lax.fori_loop(0, nsteps, count_step, 0)

    run = jnp.int32(0)
    for sb in range(NSB):
        ln = scal[S_OFFS + sb]
        scal[S_BASE + sb] = run
        aln = lax.shift_left(lax.shift_right_logical(ln + 15, 4), 4)
        scal[S_ESTEP + sb] = lax.shift_right_logical(aln, 4)
        scal[S_OFFS + sb] = run
        plsc.store_scatter(
            X, [run + ln + iota], jnp.full((16,), SENT, jnp.int32))
        run = run + aln

    def redist(u, carry):
        v = An[pl.ds(16 * u, 16)]
        kv = Ak[pl.ds(16 * u, 16)]
        valid = (16 * u + iota) < cnt
        sbv = lax.shift_right_logical(v - lo, 11)
        for sb in range(NSB):
            msb = valid & (sbv == sb)
            pc = plsc.cumsum(msb.astype(jnp.int32))
            pos = scal[S_OFFS + sb] + pc - 1
            plsc.store_scatter(X, [pos], v, mask=msb)
            plsc.store_scatter(Dk, [pos], kv, mask=msb)
            scal[S_OFFS + sb] = scal[S_OFFS + sb] + jnp.max(pc)
        return carry

    lax.fori_loop(0, nsteps, redist, 0)

    def per_block(cb, carry):
        wait_blk(cb)
        slot = jnp.bitwise_and(cb, 1)
        blo = (cb0 + cb) * 128
        sb = lax.shift_right_logical(cb, 4)
        base = scal[S_BASE + sb]
        esteps = scal[S_ESTEP + sb]

        def scan_step(u, c2):
            v = X[pl.ds(base + 16 * u, 16)]
            mb = (v >= blo) & (v < blo + 128)
            j_all = v - blo
            kvec = Dk[pl.ds(base + 16 * u, 16)]

            def any_left(mc):
                return jnp.max(mc.astype(jnp.int32)) > 0

            def extract(mc):
                ffs = plsc.all_reduce_ffs(mc)
                sel = iota == ffs
                zero = jnp.zeros((16,), jnp.int32)
                js = jnp.max(jnp.where(sel, j_all, zero))
                ks = jnp.max(jnp.where(sel, kvec, zero))
                srow = scal[S_WROW]
                sr_v = jnp.full((16,), jnp.bitwise_and(srow, 127), jnp.int32)
                for t in range(8):
                    rows = slot * 128 + 16 * t + iota
                    vals = plsc.load_gather(
                        blk, [rows, jnp.full((16,), js, jnp.int32)])
                    plsc.store_scatter(stage, [sr_v, 16 * t + iota], vals)
                gslot = jnp.bitwise_and(lax.shift_right_logical(srow, 6), 1)
                glane = jnp.bitwise_and(srow, 63)
                plsc.store_scatter(
                    kg,
                    [jnp.full((16,), gslot, jnp.int32),
                     jnp.full((16,), glane, jnp.int32)],
                    jnp.full((16,), ks, jnp.int32),
                    mask=(iota == 0),
                )
                scal[S_WROW] = srow + 1

                @pl.when(scal[S_WROW] - scal[S_FROW] >= 64)
                def _():
                    flush()

                return mc & jnp.logical_not(sel)

            lax.while_loop(any_left, extract, mb)
            return c2

        lax.fori_loop(0, esteps, scan_step, 0)

        @pl.when(cb + 2 < nb)
        def _():
            fire_blk(cb + 2)

        return carry

    lax.fori_loop(0, nb, per_block, 0)

    @pl.when(scal[S_WROW] > scal[S_FROW])
    def _():
        flush()

    def drain(i, c):
        drain_one_spill()
        return c

    lax.fori_loop(0, scal[S_NFIRE] - scal[S_NWAIT], drain, 0)


KPW = B // NW


@functools.partial(
    pl.kernel,
    mesh=_MESH,
    compiler_params=pltpu.CompilerParams(needs_layout_passes=False),
    out_type=jax.ShapeDtypeStruct((B, 128), jnp.float32),
    scratch_types=[
        pltpu.VMEM((KPW,), jnp.int32),
        pltpu.VMEM((KPW, 128), jnp.float32),
        pltpu.SemaphoreType.DMA,
    ],
)
def _permute(spill_hbm, pos_hbm, out_hbm, pos_v, rows_v, sem):
    scid = lax.axis_index("c")
    sid = lax.axis_index("s")
    wid = sid * NUM_CORES + scid
    base = wid * KPW
    pltpu.sync_copy(pos_hbm.at[pl.ds(base, KPW)], pos_v)
    pltpu.async_copy(spill_hbm.at[pos_v], rows_v, sem).wait()
    pltpu.sync_copy(rows_v, out_hbm.at[pl.ds(base, KPW), :])


def kernel(membank, n_index):
    idx = n_index.astype(jnp.int32)
    spill, pos = _scan_spill(membank, idx)
    out2 = _permute(spill, pos)
    return out2.T
